# baseline (speedup 1.0000x reference)
import jax
import jax.numpy as jnp
from jax import lax
from jax.experimental import pallas as pl
from jax.experimental.pallas import tpu as pltpu
from jax.experimental.pallas import tpu_sc as plsc

B = 16384
D = 256
TCB = 8192
NC = 1
NW = NC * 16
RW = B // NW


def _argmax_block(x_ref, idx_ref):
    xb = x_ref[...]
    m = jnp.max(xb, axis=1, keepdims=True)
    io = lax.broadcasted_iota(jnp.int32, xb.shape, 1)
    masked = jnp.where(xb == m, io, jnp.int32(D - 1))
    idx_ref[...] = jnp.min(masked, axis=1)


def _relayout_block(i_ref, q_ref, out_ref):
    n = out_ref.shape[0]
    iv = lax.broadcast_in_dim(i_ref[...], (n, 2), (0,))
    qv = lax.broadcast_in_dim(q_ref[...], (n, 2), (0,))
    col = lax.broadcasted_iota(jnp.int32, (n, 2), 1)
    out_ref[...] = jnp.where(col == 0, iv, qv)


def _lookup_body(idx_hbm, map_hbm, out_hbm, idx_v, map_v, out_v,
                 isem, osem):
    wid = lax.axis_index("s") * NC + lax.axis_index("c")
    base = wid * RW

    cp = pltpu.async_copy(idx_hbm.at[pl.ds(base, RW)], idx_v, isem)
    cpm = pltpu.async_copy(map_hbm, map_v, osem)
    cp.wait()
    cpm.wait()

    ones16 = jnp.full((16,), 1, jnp.int32)

    @plsc.parallel_loop(0, RW, step=16, unroll=4)
    def _rows(i):
        iv = idx_v[pl.ds(i, 16)] * 2
        map_i = plsc.load_gather(map_v, [iv])
        map_q = plsc.load_gather(map_v, [iv + ones16])
        out_v[pl.ds(i, 16)] = map_i
        out_v[pl.ds(RW + i, 16)] = map_q

    cpo_i = pltpu.async_copy(out_v.at[pl.ds(0, RW)],
                             out_hbm.at[pl.ds(base, RW)], osem)
    pltpu.async_copy(out_v.at[pl.ds(RW, RW)],
                     out_hbm.at[pl.ds(B + base, RW)], osem).wait()
    cpo_i.wait()


@jax.jit
def kernel(x, mapping):
    idx = pl.pallas_call(
        _argmax_block,
        grid=(B // TCB,),
        in_specs=[pl.BlockSpec((TCB, D), lambda b: (b, 0))],
        out_specs=pl.BlockSpec((TCB,), lambda b: (b,)),
        out_shape=jax.ShapeDtypeStruct((B,), jnp.int32),
    )(x)

    mesh = plsc.VectorSubcoreMesh(core_axis_name="c", subcore_axis_name="s",
                                  num_cores=NC)
    lookup = pl.kernel(
        _lookup_body,
        mesh=mesh,
        compiler_params=pltpu.CompilerParams(
            use_tc_tiling_on_sc=False, needs_layout_passes=False),
        out_type=jax.ShapeDtypeStruct((B * 2,), jnp.float32),
        scratch_types=[
            pltpu.VMEM((RW,), jnp.int32),
            pltpu.VMEM((512,), jnp.float32),
            pltpu.VMEM((RW * 2,), jnp.float32),
            pltpu.SemaphoreType.DMA,
            pltpu.SemaphoreType.DMA,
        ],
    )
    out1d = lookup(idx, mapping.reshape(-1))
    RB = 4096
    nb = B // RB
    return pl.pallas_call(
        _relayout_block,
        grid=(nb,),
        in_specs=[pl.BlockSpec((RB,), lambda b: (b,)),
                  pl.BlockSpec((RB,), lambda b: (b + nb,))],
        out_specs=pl.BlockSpec((RB, 2), lambda b: (b, 0)),
        out_shape=jax.ShapeDtypeStruct((B, 2), jnp.float32),
    )(out1d, out1d)


# speedup vs baseline: 1.0244x; 1.0244x over previous
import jax
import jax.numpy as jnp
from jax import lax
from jax.experimental import pallas as pl
from jax.experimental.pallas import tpu as pltpu
from jax.experimental.pallas import tpu_sc as plsc

B = 16384
D = 256
TCB = 4096
NC = 1
NW = NC * 16
RW = B // NW


def _argmax_block(x_ref, idx_ref):
    xb = x_ref[...]
    m = jnp.max(xb, axis=1, keepdims=True)
    io = lax.broadcasted_iota(jnp.int32, xb.shape, 1)
    masked = jnp.where(xb == m, io, jnp.int32(D - 1))
    idx_ref[...] = jnp.min(masked, axis=1)


def _relayout_block(i_ref, q_ref, out_ref):
    n = out_ref.shape[0]
    iv = lax.broadcast_in_dim(i_ref[...], (n, 2), (0,))
    qv = lax.broadcast_in_dim(q_ref[...], (n, 2), (0,))
    col = lax.broadcasted_iota(jnp.int32, (n, 2), 1)
    out_ref[...] = jnp.where(col == 0, iv, qv)


def _lookup_body(idx_hbm, map_hbm, out_hbm, idx_v, map_v, out_v,
                 isem, osem):
    wid = lax.axis_index("s") * NC + lax.axis_index("c")
    base = wid * RW

    cp = pltpu.async_copy(idx_hbm.at[pl.ds(base, RW)], idx_v, isem)
    cpm = pltpu.async_copy(map_hbm, map_v, osem)
    cp.wait()
    cpm.wait()

    ones16 = jnp.full((16,), 1, jnp.int32)

    @plsc.parallel_loop(0, RW, step=16, unroll=4)
    def _rows(i):
        iv = idx_v[pl.ds(i, 16)] * 2
        map_i = plsc.load_gather(map_v, [iv])
        map_q = plsc.load_gather(map_v, [iv + ones16])
        out_v[pl.ds(i, 16)] = map_i
        out_v[pl.ds(RW + i, 16)] = map_q

    cpo_i = pltpu.async_copy(out_v.at[pl.ds(0, RW)],
                             out_hbm.at[pl.ds(base, RW)], osem)
    pltpu.async_copy(out_v.at[pl.ds(RW, RW)],
                     out_hbm.at[pl.ds(B + base, RW)], osem).wait()
    cpo_i.wait()


@jax.jit
def kernel(x, mapping):
    idx = pl.pallas_call(
        _argmax_block,
        grid=(B // TCB,),
        in_specs=[pl.BlockSpec((TCB, D), lambda b: (b, 0))],
        out_specs=pl.BlockSpec((TCB,), lambda b: (b,)),
        out_shape=jax.ShapeDtypeStruct((B,), jnp.int32),
    )(x)

    mesh = plsc.VectorSubcoreMesh(core_axis_name="c", subcore_axis_name="s",
                                  num_cores=NC)
    lookup = pl.kernel(
        _lookup_body,
        mesh=mesh,
        compiler_params=pltpu.CompilerParams(
            use_tc_tiling_on_sc=False, needs_layout_passes=False),
        out_type=jax.ShapeDtypeStruct((B * 2,), jnp.float32),
        scratch_types=[
            pltpu.VMEM((RW,), jnp.int32),
            pltpu.VMEM((512,), jnp.float32),
            pltpu.VMEM((RW * 2,), jnp.float32),
            pltpu.SemaphoreType.DMA,
            pltpu.SemaphoreType.DMA,
        ],
    )
    out1d = lookup(idx, mapping.reshape(-1))
    RB = 4096
    nb = B // RB
    return pl.pallas_call(
        _relayout_block,
        grid=(nb,),
        in_specs=[pl.BlockSpec((RB,), lambda b: (b,)),
                  pl.BlockSpec((RB,), lambda b: (b + nb,))],
        out_specs=pl.BlockSpec((RB, 2), lambda b: (b, 0)),
        out_shape=jax.ShapeDtypeStruct((B, 2), jnp.float32),
    )(out1d, out1d)
